# baseline (speedup 1.0000x reference)
import jax
import jax.numpy as jnp
from jax.experimental import pallas as pl
from jax.experimental.pallas import tpu as pltpu

BATCH = 16
_BOUNDS = (0, 8, 32, 64)

def _chunks():
    return [(_BOUNDS[i], _BOUNDS[i + 1] - _BOUNDS[i]) for i in range(len(_BOUNDS) - 1)]

def _body(emb_any, out_any, scratch, load_sems, sems):
    cks = _chunks()
    for s, (lo, n) in enumerate(cks):
        pltpu.make_async_copy(
            emb_any.at[pl.ds(lo, n)], scratch.at[pl.ds(lo, n)], load_sems.at[s]
        ).start()
    for s, (lo, n) in enumerate(cks):
        pltpu.make_async_copy(
            emb_any.at[pl.ds(lo, n)], scratch.at[pl.ds(lo, n)], load_sems.at[s]
        ).wait()
        for b in range(BATCH):
            pltpu.make_async_copy(
                scratch.at[pl.ds(lo, n)],
                out_any.at[b, pl.ds(lo, n)],
                sems.at[b, s],
            ).start()
    for b in range(BATCH):
        for s, (lo, n) in enumerate(cks):
            pltpu.make_async_copy(
                scratch.at[pl.ds(lo, n)],
                out_any.at[b, pl.ds(lo, n)],
                sems.at[b, s],
            ).wait()

def kernel(x, grid_embedding):
    batch = x.shape[0]
    g2, f = grid_embedding.shape
    emb_t = grid_embedding.T
    nck = len(_BOUNDS) - 1
    out_t = pl.pallas_call(
        _body,
        in_specs=[pl.BlockSpec(memory_space=pl.ANY)],
        out_specs=pl.BlockSpec(memory_space=pl.ANY),
        out_shape=jax.ShapeDtypeStruct((batch, f, g2), grid_embedding.dtype),
        scratch_shapes=[
            pltpu.VMEM((f, g2), grid_embedding.dtype),
            pltpu.SemaphoreType.DMA((nck,)),
            pltpu.SemaphoreType.DMA((BATCH, nck)),
        ],
    )(emb_t)
    return jnp.transpose(out_t, (0, 2, 1))


# speedup vs baseline: 1.0155x; 1.0155x over previous
import jax
import jax.numpy as jnp
from jax.experimental import pallas as pl
from jax.experimental.pallas import tpu as pltpu

_CHUNKS = 2


def _make_body(batch, f):
    rows = f // _CHUNKS

    def body(emb_any, out_any, scratch, load_sems, sems):
        for s in range(_CHUNKS):
            pltpu.make_async_copy(
                emb_any.at[pl.ds(s * rows, rows)],
                scratch.at[pl.ds(s * rows, rows)],
                load_sems.at[s],
            ).start()
        for s in range(_CHUNKS):
            pltpu.make_async_copy(
                emb_any.at[pl.ds(s * rows, rows)],
                scratch.at[pl.ds(s * rows, rows)],
                load_sems.at[s],
            ).wait()
            for b in range(batch):
                pltpu.make_async_copy(
                    scratch.at[pl.ds(s * rows, rows)],
                    out_any.at[b, pl.ds(s * rows, rows)],
                    sems.at[b, s],
                ).start()
        for b in range(batch):
            for s in range(_CHUNKS):
                pltpu.make_async_copy(
                    scratch.at[pl.ds(s * rows, rows)],
                    out_any.at[b, pl.ds(s * rows, rows)],
                    sems.at[b, s],
                ).wait()

    return body


def kernel(x, grid_embedding):
    batch = x.shape[0]
    g2, f = grid_embedding.shape
    emb_t = grid_embedding.T
    out_t = pl.pallas_call(
        _make_body(batch, f),
        in_specs=[pl.BlockSpec(memory_space=pl.ANY)],
        out_specs=pl.BlockSpec(memory_space=pl.ANY),
        out_shape=jax.ShapeDtypeStruct((batch, f, g2), grid_embedding.dtype),
        scratch_shapes=[
            pltpu.VMEM((f, g2), grid_embedding.dtype),
            pltpu.SemaphoreType.DMA((_CHUNKS,)),
            pltpu.SemaphoreType.DMA((batch, _CHUNKS)),
        ],
    )(emb_t)
    return jnp.transpose(out_t, (0, 2, 1))


# speedup vs baseline: 1.0197x; 1.0042x over previous
import jax
import jax.numpy as jnp
from jax.experimental import pallas as pl
from jax.experimental.pallas import tpu as pltpu

_CHUNKS = 2


def _make_body(batch, f):
    bounds = (0, f // 4, f)

    def body(emb_any, out_any, scratch, load_sems, sems):
        for s in range(_CHUNKS):
            lo, n = bounds[s], bounds[s + 1] - bounds[s]
            pltpu.make_async_copy(
                emb_any.at[pl.ds(lo, n)],
                scratch.at[pl.ds(lo, n)],
                load_sems.at[s],
            ).start()
        for s in range(_CHUNKS):
            lo, n = bounds[s], bounds[s + 1] - bounds[s]
            pltpu.make_async_copy(
                emb_any.at[pl.ds(lo, n)],
                scratch.at[pl.ds(lo, n)],
                load_sems.at[s],
            ).wait()
            for b in range(batch):
                pltpu.make_async_copy(
                    scratch.at[pl.ds(lo, n)],
                    out_any.at[b, pl.ds(lo, n)],
                    sems.at[b, s],
                ).start()
        for b in range(batch):
            for s in range(_CHUNKS):
                lo, n = bounds[s], bounds[s + 1] - bounds[s]
                pltpu.make_async_copy(
                    scratch.at[pl.ds(lo, n)],
                    out_any.at[b, pl.ds(lo, n)],
                    sems.at[b, s],
                ).wait()

    return body


def kernel(x, grid_embedding):
    batch = x.shape[0]
    g2, f = grid_embedding.shape
    emb_t = grid_embedding.T
    out_t = pl.pallas_call(
        _make_body(batch, f),
        in_specs=[pl.BlockSpec(memory_space=pl.ANY)],
        out_specs=pl.BlockSpec(memory_space=pl.ANY),
        out_shape=jax.ShapeDtypeStruct((batch, f, g2), grid_embedding.dtype),
        scratch_shapes=[
            pltpu.VMEM((f, g2), grid_embedding.dtype),
            pltpu.SemaphoreType.DMA((_CHUNKS,)),
            pltpu.SemaphoreType.DMA((batch, _CHUNKS)),
        ],
    )(emb_t)
    return jnp.transpose(out_t, (0, 2, 1))
